# Initial kernel scaffold; baseline (speedup 1.0000x reference)
import jax
import jax.numpy as jnp
from jax.experimental import pallas as pl


def kernel(Zj, table):
    raise NotImplementedError("write your pallas kernel here")


# speedup vs baseline: 5.1824x; 5.1824x over previous
import functools

import jax
import jax.numpy as jnp
from jax import lax
from jax.experimental import pallas as pl
from jax.experimental.pallas import tpu as pltpu
from jax.experimental.pallas import tpu_sc as plsc

_NC = 2
_NS = 16
_NW = _NC * _NS
_L = 16


def _make_sc_gather(B, V, D, C):
    per_w = B // _NW
    n_chunks = per_w // C
    mesh = plsc.VectorSubcoreMesh(
        core_axis_name="c", subcore_axis_name="s",
        num_cores=_NC, num_subcores=_NS)

    @functools.partial(
        pl.kernel,
        out_type=jax.ShapeDtypeStruct((B * D,), jnp.float32),
        mesh=mesh,
        compiler_params=pltpu.CompilerParams(needs_layout_passes=False),
        scratch_types=[
            pltpu.VMEM((V * D,), jnp.float32),
            pltpu.VMEM((C,), jnp.int32),
            pltpu.VMEM((C * D,), jnp.float32),
        ],
    )
    def k(zj_hbm, table_hbm, out_hbm, table_v, idx_v, out_v):
        wid = lax.axis_index("s") * _NC + lax.axis_index("c")
        base = wid * per_w
        pltpu.sync_copy(table_hbm, table_v)

        lane = lax.iota(jnp.int32, _L)
        lane_d = lane * D

        def chunk_body(c, _):
            off = pl.multiple_of(base + c * C, 8)
            pltpu.sync_copy(zj_hbm.at[pl.ds(off, C)], idx_v)

            def grp(i, _):
                z = idx_v[pl.ds(i * _L, _L)]
                z_d = z * D
                out_base = i * (_L * D)
                for j in range(D):
                    col = plsc.load_gather(table_v, [z_d + j])
                    plsc.store_scatter(out_v, [lane_d + (out_base + j)], col)
                return 0

            lax.fori_loop(0, C // _L, grp, 0, unroll=False)
            o_off = pl.multiple_of((base + c * C) * D, 8)
            pltpu.sync_copy(out_v, out_hbm.at[pl.ds(o_off, C * D)])
            return 0

        lax.fori_loop(0, n_chunks, chunk_body, 0, unroll=False)

    return k


def kernel(Zj, table):
    B, = Zj.shape
    V, D = table.shape
    C = 2000
    k = _make_sc_gather(B, V, D, C)
    out = k(Zj, table.reshape(V * D))
    return out.reshape(B, D)


# speedup vs baseline: 6.3445x; 1.2242x over previous
import functools

import jax
import jax.numpy as jnp
from jax import lax
from jax.experimental import pallas as pl
from jax.experimental.pallas import tpu as pltpu
from jax.experimental.pallas import tpu_sc as plsc

_NC = 2
_NS = 16
_NW = _NC * _NS
_L = 16


def _make_sc_gather(B, V, D, C):
    per_w = B // _NW
    n_chunks = per_w // C
    mesh = plsc.VectorSubcoreMesh(
        core_axis_name="c", subcore_axis_name="s",
        num_cores=_NC, num_subcores=_NS)

    @functools.partial(
        pl.kernel,
        out_type=jax.ShapeDtypeStruct((B, D), jnp.float32),
        mesh=mesh,
        compiler_params=pltpu.CompilerParams(
            needs_layout_passes=False, use_tc_tiling_on_sc=True),
        scratch_types=[
            pltpu.VMEM((V * D,), jnp.float32),
            pltpu.VMEM((C,), jnp.int32),
            pltpu.VMEM((C, D), jnp.float32),
        ],
    )
    def k(zj_hbm, table_hbm, out_hbm, table_v, idx_v, out_v):
        wid = lax.axis_index("s") * _NC + lax.axis_index("c")
        base = wid * per_w
        pltpu.sync_copy(table_hbm, table_v)

        lane = lax.iota(jnp.int32, _L)

        def chunk_body(c, _):
            off = pl.multiple_of(base + c * C, 8)
            pltpu.sync_copy(zj_hbm.at[pl.ds(off, C)], idx_v)

            def grp(i, _):
                z = idx_v[pl.ds(i * _L, _L)]
                z_d = z * D
                row = lane + i * _L
                for j in range(D):
                    col = plsc.load_gather(table_v, [z_d + j])
                    plsc.store_scatter(
                        out_v, [row, jnp.full((_L,), j, jnp.int32)], col)
                return 0

            lax.fori_loop(0, C // _L, grp, 0, unroll=False)
            pltpu.sync_copy(out_v, out_hbm.at[pl.ds(off, C)])
            return 0

        lax.fori_loop(0, n_chunks, chunk_body, 0, unroll=False)

    return k


def kernel(Zj, table):
    B, = Zj.shape
    V, D = table.shape
    C = 800
    k = _make_sc_gather(B, V, D, C)
    return k(Zj, table.reshape(V * D))


# speedup vs baseline: 31.2872x; 4.9314x over previous
import functools

import jax
import jax.numpy as jnp
from jax import lax
from jax.experimental import pallas as pl
from jax.experimental.pallas import tpu as pltpu
from jax.experimental.pallas import tpu_sc as plsc

_NC = 2
_NS = 16
_NW = _NC * _NS
_L = 16


def _make_sc_gather(B, V, D, C):
    total_chunks = B // C
    n_full = total_chunks // _NW
    rem = total_chunks % _NW
    mesh = plsc.VectorSubcoreMesh(
        core_axis_name="c", subcore_axis_name="s",
        num_cores=_NC, num_subcores=_NS)

    @functools.partial(
        pl.kernel,
        out_type=jax.ShapeDtypeStruct((D, B), jnp.float32),
        mesh=mesh,
        compiler_params=pltpu.CompilerParams(
            needs_layout_passes=False, use_tc_tiling_on_sc=True),
        scratch_types=[
            pltpu.VMEM((V * D,), jnp.float32),
            pltpu.VMEM((C,), jnp.int32),
            pltpu.VMEM((D, C), jnp.float32),
        ],
    )
    def k(zj_hbm, table_hbm, out_hbm, table_v, idx_v, out_v):
        wid = lax.axis_index("s") * _NC + lax.axis_index("c")
        n_t = n_full + jnp.where(wid < rem, 1, 0)
        pltpu.sync_copy(table_hbm, table_v)

        lane = lax.iota(jnp.int32, _L)

        def chunk_body(t, _):
            off = pl.multiple_of((wid + t * _NW) * C, 128)
            pltpu.sync_copy(zj_hbm.at[pl.ds(off, C)], idx_v)

            def grp(i, _):
                z = idx_v[pl.ds(i * _L, _L)]
                z_d = z * D
                col_idx = lane + i * _L
                for j in range(D):
                    vals = plsc.load_gather(table_v, [z_d + j])
                    plsc.store_scatter(
                        out_v, [jnp.full((_L,), j, jnp.int32), col_idx], vals)
                return 0

            lax.fori_loop(0, C // _L, grp, 0, unroll=False)
            pltpu.sync_copy(out_v, out_hbm.at[:, pl.ds(off, C)])
            return 0

        lax.fori_loop(0, n_t, chunk_body, 0, unroll=False)

    return k


def kernel(Zj, table):
    B, = Zj.shape
    V, D = table.shape
    C = 2560
    k = _make_sc_gather(B, V, D, C)
    out_t = k(Zj, table.reshape(V * D))
    return out_t.T
